# Initial kernel scaffold; baseline (speedup 1.0000x reference)
import jax
import jax.numpy as jnp
from jax.experimental import pallas as pl


def kernel(features, edges, edge_label, params):
    raise NotImplementedError("write your pallas kernel here")


# speedup vs baseline: 1.8384x; 1.8384x over previous
import functools

import jax
import jax.numpy as jnp
from jax import lax
from jax.experimental import pallas as pl
from jax.experimental.pallas import tpu as pltpu
from jax.experimental.pallas import tpu_sc as plsc

N = 50000
E = 800000
D = 64

_NB = 2000
_CE = 2000

_SC_NC = 2
_SC_NS = 16

_EPW = E // (_SC_NC * _SC_NS)
_GCH = 128
_GFULL = _EPW // _GCH
_GTAIL = _EPW - _GFULL * _GCH

_EPT = E // _SC_NS
_SCH = 128
_SFULL = _EPT // _SCH
_STAIL = _EPT - _SFULL * _SCH
_RCH = 400
_NRC = N // _RCH
_RRC = (_NRC + _SC_NS - 1) // _SC_NS


def _leaky(x):
    return jnp.where(x >= 0, x, 0.01 * x)


def _mlp3_block(x, w1, b1, w2, b2, w3, b3):
    x = jnp.dot(x, w1, preferred_element_type=jnp.float32) + b1
    x = _leaky(x)
    x = jnp.dot(x, w2, preferred_element_type=jnp.float32) + b2
    x = _leaky(x)
    x = jnp.dot(x, w3, preferred_element_type=jnp.float32) + b3
    return x


def _wspec(shape):
    return pl.BlockSpec(shape, lambda i: (0,) * len(shape))


def _flat_params(ps):
    (w1, b1), (w2, b2), (w3, b3) = ps
    return (w1, b1.reshape(1, -1), w2, b2.reshape(1, -1), w3, b3.reshape(1, -1))


def _init_mlp(features, ws, interpret=False):
    def body(x_ref, w1, b1, w2, b2, w3, b3, o_ref):
        x = _mlp3_block(x_ref[...], w1[...], b1[...], w2[...], b2[...],
                        w3[...], b3[...])
        o_ref[...] = jnp.tanh(x)

    grid = N // _NB
    specs = [pl.BlockSpec((_NB, D), lambda i: (i, 0))]
    specs += [_wspec(w.shape) for w in ws]
    return pl.pallas_call(
        body,
        grid=(grid,),
        in_specs=specs,
        out_specs=pl.BlockSpec((_NB, D), lambda i: (i, 0)),
        out_shape=jax.ShapeDtypeStruct((N, D), jnp.float32),
        interpret=interpret,
    )(features, *ws)


def _edge_mlp(end_mats, label, ws, interpret=False):
    nm = len(end_mats)
    swap = (1, 0) if nm == 2 else (2, 3, 0, 1)

    def body(*refs):
        xs = [r[...] for r in refs[:nm]]
        lbl = refs[nm][...]
        w1, b1, w2, b2, w3, b3 = (r[...] for r in refs[nm + 1:nm + 7])
        o1, o2 = refs[nm + 7:]
        x_fwd = jnp.concatenate(xs, axis=1)
        x_bwd = jnp.concatenate([xs[j] for j in swap], axis=1)
        xx = jnp.concatenate([x_fwd, x_bwd], axis=0)
        y = _mlp3_block(xx, w1, b1, w2, b2, w3, b3)
        mask = jnp.where(lbl == 0.0, 1.0, 0.0)
        o1[...] = y[:_CE] * mask
        o2[...] = y[_CE:] * mask

    grid = E // _CE
    specs = [pl.BlockSpec((_CE, D), lambda i: (i, 0)) for _ in range(nm)]
    specs.append(pl.BlockSpec((_CE, 1), lambda i: (i, 0)))
    specs += [_wspec(w.shape) for w in ws]
    out_spec = pl.BlockSpec((_CE, D), lambda i: (i, 0))
    out_sh = jax.ShapeDtypeStruct((E, D), jnp.float32)
    return pl.pallas_call(
        body,
        grid=(grid,),
        in_specs=specs,
        out_specs=[out_spec, out_spec],
        out_shape=[out_sh, out_sh],
        interpret=interpret,
    )(*end_mats, label, *ws)


def _node_mlp(agg, h, ws, interpret=False):
    def body(a_ref, h_ref, w1, b1, w2, b2, w3, b3, o_ref):
        x = jnp.concatenate([a_ref[0], a_ref[1], h_ref[...]], axis=1)
        o_ref[...] = _mlp3_block(x, w1[...], b1[...], w2[...], b2[...],
                                 w3[...], b3[...])

    grid = N // _NB
    specs = [pl.BlockSpec((2, _NB, 32), lambda i: (0, i, 0)),
             pl.BlockSpec((_NB, D), lambda i: (i, 0))]
    specs += [_wspec(w.shape) for w in ws]
    return pl.pallas_call(
        body,
        grid=(grid,),
        in_specs=specs,
        out_specs=pl.BlockSpec((_NB, D), lambda i: (i, 0)),
        out_shape=jax.ShapeDtypeStruct((N, D), jnp.float32),
        interpret=interpret,
    )(agg, h, *ws)


def _final_mlp(agg, h1, h0, ws_n, ws_e, interpret=False):
    def body(a_ref, h1_ref, h0_ref, *refs):
        wn = [r[...] for r in refs[:6]]
        we = [r[...] for r in refs[6:12]]
        o_ref = refs[12]
        x = jnp.concatenate([a_ref[0], a_ref[1], h1_ref[...]], axis=1)
        h2 = _mlp3_block(x, *wn)
        z = jnp.concatenate([h0_ref[...], h1_ref[...], h2], axis=1)
        o_ref[...] = _mlp3_block(z, *we)

    grid = N // _NB
    specs = [pl.BlockSpec((2, _NB, 32), lambda i: (0, i, 0)),
             pl.BlockSpec((_NB, D), lambda i: (i, 0)),
             pl.BlockSpec((_NB, D), lambda i: (i, 0))]
    specs += [_wspec(w.shape) for w in ws_n]
    specs += [_wspec(w.shape) for w in ws_e]
    return pl.pallas_call(
        body,
        grid=(grid,),
        in_specs=specs,
        out_specs=pl.BlockSpec((_NB, D), lambda i: (i, 0)),
        out_shape=jax.ShapeDtypeStruct((N, D), jnp.float32),
        interpret=interpret,
    )(agg, h1, h0, *ws_n, *ws_e)


def _sc_gather(table, u_idx, v_idx):
    mesh = plsc.VectorSubcoreMesh(core_axis_name="c", subcore_axis_name="s")
    osh = jax.ShapeDtypeStruct((E, D), jnp.float32)

    @functools.partial(
        pl.kernel,
        out_type=(osh, osh),
        mesh=mesh,
        compiler_params=pltpu.CompilerParams(use_tc_tiling_on_sc=False),
        scratch_types=[
            pltpu.VMEM((_GCH,), jnp.int32),
            pltpu.VMEM((_GCH,), jnp.int32),
            pltpu.VMEM((_GCH, D), jnp.float32),
            pltpu.VMEM((_GCH, D), jnp.float32),
            pltpu.VMEM((_GTAIL,), jnp.int32),
            pltpu.VMEM((_GTAIL,), jnp.int32),
            pltpu.VMEM((_GTAIL, D), jnp.float32),
            pltpu.VMEM((_GTAIL, D), jnp.float32),
            pltpu.SemaphoreType.DMA,
            pltpu.SemaphoreType.DMA,
        ],
    )
    def gather_k(tab, uh, vh, ou, ov, iu, iv, ru, rv, tiu, tiv, tru, trv,
                 s1, s2):
        wid = lax.axis_index("s") * _SC_NC + lax.axis_index("c")
        base = wid * _EPW

        def chunk(e0, ib_u, ib_v, rb_u, rb_v, n):
            pltpu.sync_copy(uh.at[pl.ds(e0, n)], ib_u)
            pltpu.sync_copy(vh.at[pl.ds(e0, n)], ib_v)
            cu = pltpu.async_copy(tab.at[ib_u], rb_u, s1)
            cv = pltpu.async_copy(tab.at[ib_v], rb_v, s2)
            cu.wait()
            cv.wait()
            pltpu.sync_copy(rb_u, ou.at[pl.ds(e0, n), :])
            pltpu.sync_copy(rb_v, ov.at[pl.ds(e0, n), :])

        def body(i, carry):
            chunk(base + i * _GCH, iu, iv, ru, rv, _GCH)
            return carry

        lax.fori_loop(0, _GFULL, body, 0)
        chunk(base + _GFULL * _GCH, tiu, tiv, tru, trv, _GTAIL)

    return gather_k(table, u_idx, v_idx)


def _sc_scatter(m1, m2, u_idx, v_idx, zeros):
    mesh = plsc.VectorSubcoreMesh(core_axis_name="c", subcore_axis_name="s")

    @functools.partial(
        pl.kernel,
        out_type=jax.ShapeDtypeStruct((2, N, 32), jnp.float32),
        mesh=mesh,
        compiler_params=pltpu.CompilerParams(use_tc_tiling_on_sc=False),
        scratch_types=[
            pltpu.VMEM((_SCH,), jnp.int32),
            pltpu.VMEM((_SCH,), jnp.int32),
            pltpu.VMEM((_SCH, 32), jnp.float32),
            pltpu.VMEM((_SCH, 32), jnp.float32),
            pltpu.VMEM((_STAIL,), jnp.int32),
            pltpu.VMEM((_STAIL,), jnp.int32),
            pltpu.VMEM((_STAIL, 32), jnp.float32),
            pltpu.VMEM((_STAIL, 32), jnp.float32),
            pltpu.VMEM_SHARED((N, 32), jnp.float32),
        ],
    )
    def scatter_k(m1h, m2h, uh, vh, zh, aggh, iu, iv, b1, b2, tiu, tiv,
                  tb1, tb2, acc):
        c = lax.axis_index("c")
        s = lax.axis_index("s")
        col0 = c * 32

        def zbody(j, carry):
            ci = s + j * _SC_NS

            @pl.when(ci < _NRC)
            def _():
                pltpu.sync_copy(zh, acc.at[pl.ds(ci * _RCH, _RCH), :])

            return carry

        lax.fori_loop(0, _RRC, zbody, 0)
        plsc.subcore_barrier()

        base = s * _EPT

        def chunk(e0, ib_u, ib_v, mb_1, mb_2, n):
            pltpu.sync_copy(uh.at[pl.ds(e0, n)], ib_u)
            pltpu.sync_copy(vh.at[pl.ds(e0, n)], ib_v)
            pltpu.sync_copy(m1h.at[pl.ds(e0, n), pl.ds(col0, 32)], mb_1)
            pltpu.sync_copy(m2h.at[pl.ds(e0, n), pl.ds(col0, 32)], mb_2)
            pltpu.sync_copy(mb_1, acc.at[ib_u], add=True)
            pltpu.sync_copy(mb_2, acc.at[ib_v], add=True)

        def body(i, carry):
            chunk(base + i * _SCH, iu, iv, b1, b2, _SCH)
            return carry

        lax.fori_loop(0, _SFULL, body, 0)
        chunk(base + _SFULL * _SCH, tiu, tiv, tb1, tb2, _STAIL)
        plsc.subcore_barrier()

        def wbody(j, carry):
            ci = s + j * _SC_NS

            @pl.when(ci < _NRC)
            def _():
                pltpu.sync_copy(acc.at[pl.ds(ci * _RCH, _RCH), :],
                                aggh.at[c, pl.ds(ci * _RCH, _RCH), :])

            return carry

        lax.fori_loop(0, _RRC, wbody, 0)

    return scatter_k(m1, m2, u_idx, v_idx, zeros)


def kernel(features, edges, edge_label, params):
    u = edges[:, 0]
    v = edges[:, 1]
    label = edge_label.reshape(E, 1)
    zeros32 = jnp.zeros((_RCH, 32), jnp.float32)

    ws_init = _flat_params(params['f_init'])
    ws_n0 = _flat_params(params['f_n'][0])
    ws_n1 = _flat_params(params['f_n'][1])
    ws_emb = _flat_params(params['node_emb'])

    def edge_ws(t, r0):
        w1, b1, w2, b2, w3, b3 = _flat_params(params['f_ef'][t])
        a, b_, cc, dd = (w1[0:64], w1[64:128], w1[128:192], w1[192:256])
        if r0:
            w1x = jnp.concatenate([a + cc, b_ + dd], axis=0)
        else:
            w1x = jnp.concatenate([a, cc, b_, dd], axis=0)
        return (w1x, b1, w2, b2, w3, b3)

    h0 = _init_mlp(features, ws_init)
    xiu, xiv = _sc_gather(h0, u, v)

    m1, m2 = _edge_mlp([xiu, xiv], label, edge_ws(0, True))
    agg0 = _sc_scatter(m1, m2, u, v, zeros32)
    h1 = _node_mlp(agg0, h0, ws_n0)

    xhu, xhv = _sc_gather(h1, u, v)
    m1, m2 = _edge_mlp([xhu, xiu, xhv, xiv], label, edge_ws(1, False))
    agg1 = _sc_scatter(m1, m2, u, v, zeros32)

    return _final_mlp(agg1, h1, h0, ws_n1, ws_emb)


# speedup vs baseline: 2.4091x; 1.3104x over previous
import functools

import jax
import jax.numpy as jnp
from jax import lax
from jax.experimental import pallas as pl
from jax.experimental.pallas import tpu as pltpu
from jax.experimental.pallas import tpu_sc as plsc

N = 50000
E = 800000
D = 64

_NB = 2000
_CE = 2000

_SC_NC = 2
_SC_NS = 16

_EPW = E // (_SC_NC * _SC_NS)
_GCH = 128
_GFULL = _EPW // _GCH
_GTAIL = _EPW - _GFULL * _GCH

_EPT = E // _SC_NS
_SCH = 128
_SFULL = _EPT // _SCH
_STAIL = _EPT - _SFULL * _SCH
_RCH = 400
_NRC = N // _RCH
_RRC = (_NRC + _SC_NS - 1) // _SC_NS


def _leaky(x):
    return jnp.where(x >= 0, x, 0.01 * x)


def _mlp3_block(x, w1, b1, w2, b2, w3, b3):
    x = jnp.dot(x, w1, preferred_element_type=jnp.float32) + b1
    x = _leaky(x)
    x = jnp.dot(x, w2, preferred_element_type=jnp.float32) + b2
    x = _leaky(x)
    x = jnp.dot(x, w3, preferred_element_type=jnp.float32) + b3
    return x


def _wspec(shape):
    return pl.BlockSpec(shape, lambda i: (0,) * len(shape))


def _flat_params(ps):
    (w1, b1), (w2, b2), (w3, b3) = ps
    return (w1, b1.reshape(1, -1), w2, b2.reshape(1, -1), w3, b3.reshape(1, -1))


def _init_mlp(features, ws, interpret=False):
    def body(x_ref, w1, b1, w2, b2, w3, b3, o_ref):
        x = _mlp3_block(x_ref[...], w1[...], b1[...], w2[...], b2[...],
                        w3[...], b3[...])
        o_ref[...] = jnp.tanh(x)

    grid = N // _NB
    specs = [pl.BlockSpec((_NB, D), lambda i: (i, 0))]
    specs += [_wspec(w.shape) for w in ws]
    return pl.pallas_call(
        body,
        grid=(grid,),
        in_specs=specs,
        out_specs=pl.BlockSpec((_NB, D), lambda i: (i, 0)),
        out_shape=jax.ShapeDtypeStruct((N, D), jnp.float32),
        interpret=interpret,
    )(features, *ws)


def _edge_mlp(end_mats, label, ws, interpret=False):
    nm = len(end_mats)
    swap = (1, 0) if nm == 2 else (2, 3, 0, 1)

    def body(*refs):
        xs = [r[...] for r in refs[:nm]]
        lbl = refs[nm][...]
        w1, b1, w2, b2, w3, b3 = (r[...] for r in refs[nm + 1:nm + 7])
        o1, o2 = refs[nm + 7:]
        x_fwd = jnp.concatenate(xs, axis=1)
        x_bwd = jnp.concatenate([xs[j] for j in swap], axis=1)
        xx = jnp.concatenate([x_fwd, x_bwd], axis=0)
        y = _mlp3_block(xx, w1, b1, w2, b2, w3, b3)
        mask = jnp.where(lbl == 0.0, 1.0, 0.0)
        o1[...] = y[:_CE] * mask
        o2[...] = y[_CE:] * mask

    grid = E // _CE
    specs = [pl.BlockSpec((_CE, D), lambda i: (i, 0)) for _ in range(nm)]
    specs.append(pl.BlockSpec((_CE, 1), lambda i: (i, 0)))
    specs += [_wspec(w.shape) for w in ws]
    out_spec = pl.BlockSpec((_CE, D), lambda i: (i, 0))
    out_sh = jax.ShapeDtypeStruct((E, D), jnp.float32)
    return pl.pallas_call(
        body,
        grid=(grid,),
        in_specs=specs,
        out_specs=[out_spec, out_spec],
        out_shape=[out_sh, out_sh],
        interpret=interpret,
    )(*end_mats, label, *ws)


def _node_mlp(agg, h, ws, interpret=False):
    def body(a_ref, h_ref, w1, b1, w2, b2, w3, b3, o_ref):
        x = jnp.concatenate([a_ref[0], a_ref[1], h_ref[...]], axis=1)
        o_ref[...] = _mlp3_block(x, w1[...], b1[...], w2[...], b2[...],
                                 w3[...], b3[...])

    grid = N // _NB
    specs = [pl.BlockSpec((2, _NB, 32), lambda i: (0, i, 0)),
             pl.BlockSpec((_NB, D), lambda i: (i, 0))]
    specs += [_wspec(w.shape) for w in ws]
    return pl.pallas_call(
        body,
        grid=(grid,),
        in_specs=specs,
        out_specs=pl.BlockSpec((_NB, D), lambda i: (i, 0)),
        out_shape=jax.ShapeDtypeStruct((N, D), jnp.float32),
        interpret=interpret,
    )(agg, h, *ws)


def _final_mlp(agg, h1, h0, ws_n, ws_e, interpret=False):
    def body(a_ref, h1_ref, h0_ref, *refs):
        wn = [r[...] for r in refs[:6]]
        we = [r[...] for r in refs[6:12]]
        o_ref = refs[12]
        x = jnp.concatenate([a_ref[0], a_ref[1], h1_ref[...]], axis=1)
        h2 = _mlp3_block(x, *wn)
        z = jnp.concatenate([h0_ref[...], h1_ref[...], h2], axis=1)
        o_ref[...] = _mlp3_block(z, *we)

    grid = N // _NB
    specs = [pl.BlockSpec((2, _NB, 32), lambda i: (0, i, 0)),
             pl.BlockSpec((_NB, D), lambda i: (i, 0)),
             pl.BlockSpec((_NB, D), lambda i: (i, 0))]
    specs += [_wspec(w.shape) for w in ws_n]
    specs += [_wspec(w.shape) for w in ws_e]
    return pl.pallas_call(
        body,
        grid=(grid,),
        in_specs=specs,
        out_specs=pl.BlockSpec((_NB, D), lambda i: (i, 0)),
        out_shape=jax.ShapeDtypeStruct((N, D), jnp.float32),
        interpret=interpret,
    )(agg, h1, h0, *ws_n, *ws_e)


def _sc_gather(table, u_idx, v_idx):
    mesh = plsc.VectorSubcoreMesh(core_axis_name="c", subcore_axis_name="s")
    osh = jax.ShapeDtypeStruct((E, D), jnp.float32)

    @functools.partial(
        pl.kernel,
        out_type=(osh, osh),
        mesh=mesh,
        compiler_params=pltpu.CompilerParams(use_tc_tiling_on_sc=False),
        scratch_types=[
            pltpu.VMEM((_EPW,), jnp.int32),
            pltpu.VMEM((_EPW,), jnp.int32),
            pltpu.VMEM((_GCH, D), jnp.float32),
            pltpu.VMEM((_GCH, D), jnp.float32),
            pltpu.VMEM((_GCH, D), jnp.float32),
            pltpu.VMEM((_GCH, D), jnp.float32),
            pltpu.SemaphoreType.DMA,
            pltpu.SemaphoreType.DMA,
            pltpu.SemaphoreType.DMA,
            pltpu.SemaphoreType.DMA,
        ],
    )
    def gather_k(tab, uh, vh, ou, ov, iua, iva, ru0, rv0, ru1, rv1,
                 sg0, sg1, sw0, sw1):
        wid = lax.axis_index("s") * _SC_NC + lax.axis_index("c")
        base = wid * _EPW
        pltpu.sync_copy(uh.at[pl.ds(base, _EPW)], iua)
        pltpu.sync_copy(vh.at[pl.ds(base, _EPW)], iva)
        slots = ((ru0, rv0, sg0, sw0), (ru1, rv1, sg1, sw1))

        def start(i, slot, n):
            ru, rv, sg, _ = slots[slot]
            er = i * _GCH
            pltpu.async_copy(tab.at[iua.at[pl.ds(er, n)]],
                             ru.at[pl.ds(0, n), :], sg)
            pltpu.async_copy(tab.at[iva.at[pl.ds(er, n)]],
                             rv.at[pl.ds(0, n), :], sg)

        def flush(i, slot, n):
            ru, rv, sg, sw = slots[slot]
            er = i * _GCH
            pltpu.make_async_copy(tab.at[iua.at[pl.ds(er, n)]],
                                  ru.at[pl.ds(0, n), :], sg).wait()
            pltpu.make_async_copy(tab.at[iva.at[pl.ds(er, n)]],
                                  rv.at[pl.ds(0, n), :], sg).wait()
            pltpu.async_copy(ru.at[pl.ds(0, n), :],
                             ou.at[pl.ds(base + er, n), :], sw)
            pltpu.async_copy(rv.at[pl.ds(0, n), :],
                             ov.at[pl.ds(base + er, n), :], sw)

        def drain_wb(i, slot, n):
            ru, rv, _, sw = slots[slot]
            er = i * _GCH
            pltpu.make_async_copy(ru.at[pl.ds(0, n), :],
                                  ou.at[pl.ds(base + er, n), :], sw).wait()
            pltpu.make_async_copy(rv.at[pl.ds(0, n), :],
                                  ov.at[pl.ds(base + er, n), :], sw).wait()

        npair = _GFULL // 2

        def body(j, carry):
            @pl.when(j > 0)
            def _():
                drain_wb(2 * j - 2, 0, _GCH)
                drain_wb(2 * j - 1, 1, _GCH)

            start(2 * j, 0, _GCH)
            start(2 * j + 1, 1, _GCH)
            flush(2 * j, 0, _GCH)
            flush(2 * j + 1, 1, _GCH)
            return carry

        lax.fori_loop(0, npair, body, 0)
        drain_wb(2 * npair - 2, 0, _GCH)
        drain_wb(2 * npair - 1, 1, _GCH)
        start(_GFULL - 1, 0, _GCH)
        start(_GFULL, 1, _GTAIL)
        flush(_GFULL - 1, 0, _GCH)
        flush(_GFULL, 1, _GTAIL)
        drain_wb(_GFULL - 1, 0, _GCH)
        drain_wb(_GFULL, 1, _GTAIL)

    return gather_k(table, u_idx, v_idx)


def _sc_scatter(m1, m2, u_idx, v_idx, zeros):
    mesh = plsc.VectorSubcoreMesh(core_axis_name="c", subcore_axis_name="s")

    @functools.partial(
        pl.kernel,
        out_type=jax.ShapeDtypeStruct((2, N, 32), jnp.float32),
        mesh=mesh,
        compiler_params=pltpu.CompilerParams(use_tc_tiling_on_sc=False),
        scratch_types=[
            pltpu.VMEM((_SCH,), jnp.int32),
            pltpu.VMEM((_SCH,), jnp.int32),
            pltpu.VMEM((_SCH,), jnp.int32),
            pltpu.VMEM((_SCH,), jnp.int32),
            pltpu.VMEM((_SCH, 32), jnp.float32),
            pltpu.VMEM((_SCH, 32), jnp.float32),
            pltpu.VMEM((_SCH, 32), jnp.float32),
            pltpu.VMEM((_SCH, 32), jnp.float32),
            pltpu.VMEM((_STAIL,), jnp.int32),
            pltpu.VMEM((_STAIL,), jnp.int32),
            pltpu.VMEM((_STAIL, 32), jnp.float32),
            pltpu.VMEM((_STAIL, 32), jnp.float32),
            pltpu.VMEM_SHARED((N, 32), jnp.float32),
            pltpu.SemaphoreType.DMA,
            pltpu.SemaphoreType.DMA,
            pltpu.SemaphoreType.DMA,
            pltpu.SemaphoreType.DMA,
        ],
    )
    def scatter_k(m1h, m2h, uh, vh, zh, aggh, iu0, iv0, iu1, iv1,
                  b10, b20, b11, b21, tiu, tiv, tb1, tb2, acc,
                  sl0, sl1, ss0, ss1):
        c = lax.axis_index("c")
        s = lax.axis_index("s")
        col0 = c * 32

        def zbody(j, carry):
            ci = s + j * _SC_NS

            @pl.when(ci < _NRC)
            def _():
                pltpu.sync_copy(zh, acc.at[pl.ds(ci * _RCH, _RCH), :])

            return carry

        lax.fori_loop(0, _RRC, zbody, 0)
        plsc.subcore_barrier()

        base = s * _EPT
        slots = ((iu0, iv0, b10, b20, sl0, ss0),
                 (iu1, iv1, b11, b21, sl1, ss1))

        def start(i, slot):
            iu, iv, b1, b2, sl, _ = slots[slot]
            e0 = base + i * _SCH
            pltpu.async_copy(uh.at[pl.ds(e0, _SCH)], iu, sl)
            pltpu.async_copy(vh.at[pl.ds(e0, _SCH)], iv, sl)
            pltpu.async_copy(m1h.at[pl.ds(e0, _SCH), pl.ds(col0, 32)],
                             b1, sl)
            pltpu.async_copy(m2h.at[pl.ds(e0, _SCH), pl.ds(col0, 32)],
                             b2, sl)

        def flush(i, slot):
            iu, iv, b1, b2, sl, ss = slots[slot]
            e0 = base + i * _SCH
            pltpu.make_async_copy(uh.at[pl.ds(e0, _SCH)], iu, sl).wait()
            pltpu.make_async_copy(vh.at[pl.ds(e0, _SCH)], iv, sl).wait()
            pltpu.make_async_copy(m1h.at[pl.ds(e0, _SCH), pl.ds(col0, 32)],
                                  b1, sl).wait()
            pltpu.make_async_copy(m2h.at[pl.ds(e0, _SCH), pl.ds(col0, 32)],
                                  b2, sl).wait()
            pltpu.async_copy(b1, acc.at[iu], ss, add=True)
            pltpu.async_copy(b2, acc.at[iv], ss, add=True)

        def drain_sc(slot):
            iu, iv, b1, b2, _, ss = slots[slot]
            pltpu.make_async_copy(b1, acc.at[iu], ss).wait()
            pltpu.make_async_copy(b2, acc.at[iv], ss).wait()

        npair = _SFULL // 2

        def body(j, carry):
            @pl.when(j > 0)
            def _():
                drain_sc(0)
                drain_sc(1)

            start(2 * j, 0)
            start(2 * j + 1, 1)
            flush(2 * j, 0)
            flush(2 * j + 1, 1)
            return carry

        lax.fori_loop(0, npair, body, 0)
        drain_sc(0)
        drain_sc(1)
        e0 = base + _SFULL * _SCH
        pltpu.sync_copy(uh.at[pl.ds(e0, _STAIL)], tiu)
        pltpu.sync_copy(vh.at[pl.ds(e0, _STAIL)], tiv)
        pltpu.sync_copy(m1h.at[pl.ds(e0, _STAIL), pl.ds(col0, 32)], tb1)
        pltpu.sync_copy(m2h.at[pl.ds(e0, _STAIL), pl.ds(col0, 32)], tb2)
        pltpu.sync_copy(tb1, acc.at[tiu], add=True)
        pltpu.sync_copy(tb2, acc.at[tiv], add=True)
        plsc.subcore_barrier()

        def wbody(j, carry):
            ci = s + j * _SC_NS

            @pl.when(ci < _NRC)
            def _():
                pltpu.sync_copy(acc.at[pl.ds(ci * _RCH, _RCH), :],
                                aggh.at[c, pl.ds(ci * _RCH, _RCH), :])

            return carry

        lax.fori_loop(0, _RRC, wbody, 0)

    return scatter_k(m1, m2, u_idx, v_idx, zeros)


def kernel(features, edges, edge_label, params):
    u = edges[:, 0]
    v = edges[:, 1]
    label = edge_label.reshape(E, 1)
    zeros32 = jnp.zeros((_RCH, 32), jnp.float32)

    ws_init = _flat_params(params['f_init'])
    ws_n0 = _flat_params(params['f_n'][0])
    ws_n1 = _flat_params(params['f_n'][1])
    ws_emb = _flat_params(params['node_emb'])

    def edge_ws(t, r0):
        w1, b1, w2, b2, w3, b3 = _flat_params(params['f_ef'][t])
        a, b_, cc, dd = (w1[0:64], w1[64:128], w1[128:192], w1[192:256])
        if r0:
            w1x = jnp.concatenate([a + cc, b_ + dd], axis=0)
        else:
            w1x = jnp.concatenate([a, cc, b_, dd], axis=0)
        return (w1x, b1, w2, b2, w3, b3)

    h0 = _init_mlp(features, ws_init)
    xiu, xiv = _sc_gather(h0, u, v)

    m1, m2 = _edge_mlp([xiu, xiv], label, edge_ws(0, True))
    agg0 = _sc_scatter(m1, m2, u, v, zeros32)
    h1 = _node_mlp(agg0, h0, ws_n0)

    xhu, xhv = _sc_gather(h1, u, v)
    m1, m2 = _edge_mlp([xhu, xiu, xhv, xiv], label, edge_ws(1, False))
    agg1 = _sc_scatter(m1, m2, u, v, zeros32)

    return _final_mlp(agg1, h1, h0, ws_n1, ws_emb)
